# Initial kernel scaffold; baseline (speedup 1.0000x reference)
import jax
import jax.numpy as jnp
from jax.experimental import pallas as pl


def kernel(x, emb_table):
    raise NotImplementedError("write your pallas kernel here")


# speedup vs baseline: 1.5637x; 1.5637x over previous
import jax
import jax.numpy as jnp
from jax import lax
from jax.experimental import pallas as pl
from jax.experimental.pallas import tpu as pltpu
from jax.experimental.pallas import tpu_sc as plsc

N = 100000
VOCAB = 21
D = 128
OUT_W = 130
CH = 160
NSTEPS = N // CH
NC, NS, L = 2, 16, 16
NW = NC * NS
MAX_K = -(-NSTEPS // NW)


def _body(x_hbm, rot_hbm, out_hbm, x_v, idx_v, g_v, t_v, sem):
    wid = lax.axis_index("s") * NC + lax.axis_index("c")
    lanes = lax.iota(jnp.int32, L)
    zero = jnp.zeros((L,), jnp.int32)
    one = zero + 1
    two = zero + 2

    for k in range(MAX_K):
        step = k * NW + wid

        @pl.when(step < NSTEPS)
        def _():
            base = step * CH
            pltpu.sync_copy(x_hbm.at[pl.ds(base, CH), :], x_v)
            for t in range(CH // L):
                rvec = lanes + t * L
                idx_v[pl.ds(t * L, L)] = plsc.load_gather(x_v, [rvec, two])
            pltpu.async_copy(rot_hbm.at[idx_v], g_v, sem).wait()
            for t in range(CH // L):
                rvec = lanes + t * L
                e126 = plsc.load_gather(g_v, [rvec, zero])
                e127 = plsc.load_gather(g_v, [rvec, one])
                plsc.store_scatter(t_v, [rvec, zero], e126)
                plsc.store_scatter(t_v, [rvec, one], e127)
                a0 = plsc.load_gather(x_v, [rvec, zero])
                a1 = plsc.load_gather(x_v, [rvec, one])
                plsc.store_scatter(g_v, [rvec, zero], a0.astype(jnp.float32))
                plsc.store_scatter(g_v, [rvec, one], a1.astype(jnp.float32))
            pltpu.sync_copy(g_v, out_hbm.at[pl.ds(base, CH), pl.ds(0, D)])
            pltpu.sync_copy(t_v, out_hbm.at[pl.ds(base, CH), pl.ds(D, 2)])


@jax.jit
def _run(x, emb_table):
    rot = jnp.concatenate([emb_table[:, D - 2:], emb_table[:, : D - 2]], axis=1)
    mesh = plsc.VectorSubcoreMesh(core_axis_name="c", subcore_axis_name="s")
    f = pl.kernel(
        _body,
        out_type=jax.ShapeDtypeStruct((N, OUT_W), jnp.float32),
        mesh=mesh,
        scratch_types=[
            pltpu.VMEM((CH, 3), jnp.int32),
            pltpu.VMEM((CH,), jnp.int32),
            pltpu.VMEM((CH, D), jnp.float32),
            pltpu.VMEM((CH, 2), jnp.float32),
            pltpu.SemaphoreType.DMA,
        ],
        compiler_params=pltpu.CompilerParams(needs_layout_passes=False),
    )
    return f(x, rot)


def kernel(x, emb_table):
    return _run(x, emb_table)
